# baseline (speedup 1.0000x reference)
import functools

import numpy as np
import jax
import jax.numpy as jnp
from jax import lax
from jax.experimental import pallas as pl
from jax.experimental.pallas import tpu as pltpu
from jax.experimental.pallas import tpu_sc as plsc

DIM = 16
NPROP = 26
NOBJ = 21
NVAL = 100000
BATCH = 1024
ROWS = BATCH * NOBJ * NPROP
FLAT = NOBJ * NPROP * DIM

NC, NS, L = 2, 16, 16
NW = NC * NS
RPW = ROWS // NW
STEP = 96
SPW = RPW // STEP
KSTEP = 13
NSUP = SPW // KSTEP
SUP = KSTEP * STEP


def _sc_gather(x3d, table):
    mesh = plsc.VectorSubcoreMesh(
        core_axis_name="c", subcore_axis_name="s",
        num_cores=NC, num_subcores=NS)

    @functools.partial(
        pl.kernel,
        out_type=jax.ShapeDtypeStruct((ROWS, DIM), jnp.float32),
        mesh=mesh,
        scratch_types=[
            pltpu.VMEM((SPW, STEP), jnp.int32),
            pltpu.VMEM((SUP, DIM), jnp.float32),
            pltpu.SemaphoreType.DMA,
        ],
        compiler_params=pltpu.CompilerParams(use_tc_tiling_on_sc=False),
    )
    def k(x_hbm, table_hbm, out_hbm, idx_v, rows_v, sem):
        wid = lax.axis_index("s") * NC + lax.axis_index("c")
        row_base = wid * RPW
        pltpu.sync_copy(x_hbm.at[wid], idx_v)

        lanes = lax.iota(jnp.int32, L)

        def to_idx(i, carry):
            for j in range(STEP // L):
                r0 = row_base + i * STEP + j * L
                prop = (r0 + lanes) % NPROP
                v = idx_v[i, pl.ds(j * L, L)]
                idx_v[i, pl.ds(j * L, L)] = v + prop * NVAL
            return carry

        lax.fori_loop(0, SPW, to_idx, 0)

        def sup(s, carry):
            cps = [
                pltpu.async_copy(
                    table_hbm.at[idx_v.at[s * KSTEP + j]],
                    rows_v.at[pl.ds(j * STEP, STEP)],
                    sem)
                for j in range(KSTEP)
            ]
            for c in cps:
                c.wait()
            pltpu.sync_copy(rows_v, out_hbm.at[pl.ds(row_base + s * SUP, SUP)])
            return carry

        lax.fori_loop(0, NSUP, sup, 0)

    return k(x3d, table)


_EG = (np.arange(NOBJ * NPROP)[:, None] // NPROP
       == np.arange(NOBJ)[None, :]).astype(np.float32)
_E16 = (np.arange(NOBJ)[:, None]
        == np.arange(FLAT)[None, :] // (NPROP * DIM)).astype(np.float32)
_E546 = _EG.T.copy()

_B_BLK = 128


def _tc_finish(raw2, x2, pattern, absent_t):
    grid = (BATCH // _B_BLK,)

    def body(raw_ref, x_ref, pat_ref, abs_ref, eg_ref, e16_ref, e546_ref,
             out_ref, pad_ref):
        xf = x_ref[...].astype(jnp.float32)
        sums = jnp.dot(xf, eg_ref[...], preferred_element_type=jnp.float32)
        padf = (sums == 0.0).astype(jnp.float32)
        m16 = jnp.dot(padf, e16_ref[...], preferred_element_type=jnp.float32)
        m546 = jnp.dot(padf, e546_ref[...], preferred_element_type=jnp.float32)
        emb = raw_ref[...] + pat_ref[...]
        out_ref[...] = emb * (1.0 - m16) + abs_ref[...] * m16
        pad_ref[...] = m546 > 0.5

    out2, padflat = pl.pallas_call(
        body,
        grid=grid,
        in_specs=[
            pl.BlockSpec((_B_BLK, FLAT), lambda i: (i, 0)),
            pl.BlockSpec((_B_BLK, NOBJ * NPROP), lambda i: (i, 0)),
            pl.BlockSpec((1, FLAT), lambda i: (0, 0)),
            pl.BlockSpec((1, FLAT), lambda i: (0, 0)),
            pl.BlockSpec((NOBJ * NPROP, NOBJ), lambda i: (0, 0)),
            pl.BlockSpec((NOBJ, FLAT), lambda i: (0, 0)),
            pl.BlockSpec((NOBJ, NOBJ * NPROP), lambda i: (0, 0)),
        ],
        out_specs=[
            pl.BlockSpec((_B_BLK, FLAT), lambda i: (i, 0)),
            pl.BlockSpec((_B_BLK, NOBJ * NPROP), lambda i: (i, 0)),
        ],
        out_shape=[
            jax.ShapeDtypeStruct((BATCH, FLAT), jnp.float32),
            jax.ShapeDtypeStruct((BATCH, NOBJ * NPROP), jnp.bool_),
        ],
    )(raw2, x2, pattern, absent_t, jnp.asarray(_EG), jnp.asarray(_E16),
      jnp.asarray(_E546))
    return out2, padflat


def kernel(table, mark_features, mark_objects, mark_absent, x):
    x3d = x.reshape(NW, SPW, STEP)
    raw = _sc_gather(x3d, table)
    raw2 = raw.reshape(BATCH, FLAT)

    pattern = (mark_objects.reshape(NOBJ, 1, DIM)
               + mark_features.reshape(1, NPROP, DIM)).reshape(1, FLAT)
    absent_t = jnp.tile(mark_absent.reshape(1, DIM), (1, NOBJ * NPROP))
    x2 = x.reshape(BATCH, NOBJ * NPROP)

    out2, padflat = _tc_finish(raw2, x2, pattern, absent_t)
    return out2.reshape(BATCH, NOBJ * NPROP, DIM), padflat
